# baseline (speedup 1.0000x reference)
import functools

import jax
import jax.numpy as jnp
from jax import lax
from jax.experimental import pallas as pl
from jax.experimental.pallas import tpu as pltpu
from jax.experimental.pallas import tpu_sc as plsc

_NE = 1024
_D = 64
_HW = 576
_B = 64
_N = _B * _HW

_NC = 2
_NS = 16
_NW = _NC * _NS
_BPT = _B // _NW
_RG = 4
_RPS = _RG * _HW


def _vq_body(z_ref, emb_ref, embm2_ref, idx_ref, loss_ref):
    x = z_ref[...].reshape(_RPS, _D)
    emb = emb_ref[...]
    a2 = jnp.sum(x * x, axis=1, keepdims=True)
    b2 = jnp.sum(emb * emb, axis=1)[None, :]
    ab2 = lax.dot_general(x, embm2_ref[...], (((1,), (1,)), ((), ())),
                          preferred_element_type=jnp.float32)
    sq = (a2 + b2) + ab2
    d = jnp.sqrt(jnp.maximum(sq, 0.0))
    dmin = jnp.min(d, axis=1, keepdims=True)
    ji = lax.broadcasted_iota(jnp.int32, (_RPS, _NE), 1)
    idx = jnp.min(jnp.where(d == dmin, ji, jnp.int32(2**30)), axis=1)
    idx_ref[0, 0, :] = idx
    part = jnp.sum(dmin * dmin)
    @pl.when(pl.program_id(0) == 0)
    def _():
        loss_ref[0, 0] = 0.0
    loss_ref[0, 0] += part


_sc_mesh = plsc.VectorSubcoreMesh(core_axis_name="c", subcore_axis_name="s")


@functools.partial(
    pl.kernel,
    mesh=_sc_mesh,
    out_type=jax.ShapeDtypeStruct((_B, _HW, _D), jnp.float32),
    scratch_types=[
        pltpu.VMEM((_HW,), jnp.int32),
        pltpu.VMEM((_HW, _D), jnp.float32),
        pltpu.SemaphoreType.DMA,
    ],
    compiler_params=pltpu.CompilerParams(use_tc_tiling_on_sc=False),
)
def _gather_rows(idx_hbm, tab_hbm, out_hbm, idx_v, rows_v, sem):
    wid = lax.axis_index("s") * _NC + lax.axis_index("c")
    for bl in range(_BPT):
        b = wid * _BPT + bl
        pltpu.sync_copy(idx_hbm.at[pl.ds(b * _HW, _HW)], idx_v)
        pltpu.async_copy(tab_hbm.at[idx_v], rows_v, sem).wait()
        pltpu.sync_copy(rows_v, out_hbm.at[b])


@jax.jit
def kernel(z, embeddings):
    zf = z.reshape(_B // _RG, _RPS, _D)
    idx3, loss_acc = pl.pallas_call(
        _vq_body,
        grid=(_B // _RG,),
        in_specs=[
            pl.BlockSpec((1, _RPS, _D), lambda i: (i, 0, 0)),
            pl.BlockSpec((_NE, _D), lambda i: (0, 0)),
            pl.BlockSpec((_NE, _D), lambda i: (0, 0)),
        ],
        out_specs=[
            pl.BlockSpec((1, 1, _RPS), lambda i: (i, 0, 0)),
            pl.BlockSpec(memory_space=pltpu.SMEM, block_shape=(1, 1),
                         index_map=lambda i: (0, 0)),
        ],
        out_shape=[
            jax.ShapeDtypeStruct((_B // _RG, 1, _RPS), jnp.int32),
            jax.ShapeDtypeStruct((1, 1), jnp.float32),
        ],
        compiler_params=pltpu.CompilerParams(
            dimension_semantics=("arbitrary",)),
    )(zf, embeddings, embeddings * jnp.float32(-2.0))
    qn = _gather_rows(idx3.reshape(_N), embeddings)
    quantized = qn.transpose(0, 2, 1).reshape(_B, _D, 24, 24)
    indices = idx3.reshape(_B, 1, 24, 24)
    loss = (loss_acc[0, 0] / jnp.float32(_N * _D)) * jnp.float32(1.25)
    return quantized, indices, loss


# speedup vs baseline: 1.0463x; 1.0463x over previous
import functools

import jax
import jax.numpy as jnp
from jax import lax
from jax.experimental import pallas as pl
from jax.experimental.pallas import tpu as pltpu
from jax.experimental.pallas import tpu_sc as plsc

_NE = 1024
_DP = 128
_D = 64
_HW = 576
_B = 64
_N = _B * _HW

_NC = 2
_NS = 16
_NW = _NC * _NS
_BPT = _B // _NW
_RG = 4
_RPS = _RG * _HW


def _vq_body(z_ref, emb_ref, embm2_ref, idx_ref, loss_ref):
    x = z_ref[...].reshape(_RPS, _D)
    emb = emb_ref[...]
    a2 = jnp.sum(x * x, axis=1, keepdims=True)
    b2 = jnp.sum(emb * emb, axis=1)[None, :]
    ab2 = lax.dot_general(x, embm2_ref[...], (((1,), (1,)), ((), ())),
                          preferred_element_type=jnp.float32)
    sq = (a2 + b2) + ab2
    d = jnp.sqrt(jnp.maximum(sq, 0.0))
    dmin = jnp.min(d, axis=1, keepdims=True)
    ji = lax.broadcasted_iota(jnp.int32, (_RPS, _NE), 1)
    idx = jnp.min(jnp.where(d == dmin, ji, jnp.int32(2**30)), axis=1)
    idx_ref[0, 0, :] = idx
    part = jnp.sum(dmin * dmin)
    @pl.when(pl.program_id(0) == 0)
    def _():
        loss_ref[0, 0] = 0.0
    loss_ref[0, 0] += part


_sc_mesh = plsc.VectorSubcoreMesh(core_axis_name="c", subcore_axis_name="s")


_CH = 288
_NCH = (_BPT * _HW) // _CH


@functools.partial(
    pl.kernel,
    mesh=_sc_mesh,
    out_type=jax.ShapeDtypeStruct((_N // _CH, _CH, _DP), jnp.float32),
    scratch_types=[
        pltpu.VMEM((_CH,), jnp.int32),
        pltpu.VMEM((_CH,), jnp.int32),
        pltpu.VMEM((_CH, _DP), jnp.float32),
        pltpu.VMEM((_CH, _DP), jnp.float32),
        pltpu.SemaphoreType.DMA,
        pltpu.SemaphoreType.DMA,
    ],
)
def _gather_rows(idx_hbm, tab_hbm, out_hbm, idx_v0, idx_v1, rows_v0,
                 rows_v1, sem0, sem1):
    wid = lax.axis_index("s") * _NC + lax.axis_index("c")
    c0 = wid * _NCH
    idx_bufs = (idx_v0, idx_v1)
    row_bufs = (rows_v0, rows_v1)
    sems = (sem0, sem1)
    pltpu.sync_copy(idx_hbm.at[pl.ds(c0 * _CH, _CH)], idx_v0)
    g_prev = pltpu.async_copy(tab_hbm.at[idx_v0], rows_v0, sem0)
    for k in range(_NCH):
        nxt = (k + 1) % 2
        if k + 1 < _NCH:
            pltpu.sync_copy(
                idx_hbm.at[pl.ds((c0 + k + 1) * _CH, _CH)], idx_bufs[nxt])
        g_prev.wait()
        if k + 1 < _NCH:
            g_next = pltpu.async_copy(
                tab_hbm.at[idx_bufs[nxt]], row_bufs[nxt], sems[nxt])
        pltpu.sync_copy(row_bufs[k % 2], out_hbm.at[c0 + k])
        if k + 1 < _NCH:
            g_prev = g_next


@jax.jit
def kernel(z, embeddings):
    zf = z.reshape(_B // _RG, _RPS, _D)
    idx3, loss_acc = pl.pallas_call(
        _vq_body,
        grid=(_B // _RG,),
        in_specs=[
            pl.BlockSpec((1, _RPS, _D), lambda i: (i, 0, 0)),
            pl.BlockSpec((_NE, _D), lambda i: (0, 0)),
            pl.BlockSpec((_NE, _D), lambda i: (0, 0)),
        ],
        out_specs=[
            pl.BlockSpec((1, 1, _RPS), lambda i: (i, 0, 0)),
            pl.BlockSpec(memory_space=pltpu.SMEM, block_shape=(1, 1),
                         index_map=lambda i: (0, 0)),
        ],
        out_shape=[
            jax.ShapeDtypeStruct((_B // _RG, 1, _RPS), jnp.int32),
            jax.ShapeDtypeStruct((1, 1), jnp.float32),
        ],
        compiler_params=pltpu.CompilerParams(
            dimension_semantics=("arbitrary",)),
    )(zf, embeddings, embeddings * jnp.float32(-2.0))
    tab_pad = jnp.concatenate(
        [embeddings, jnp.zeros((_NE, _DP - _D), jnp.float32)], axis=1)
    qp = _gather_rows(idx3.reshape(_N), tab_pad)
    quantized = qp[:, :, :_D].reshape(_B, _HW, _D).transpose(
        0, 2, 1).reshape(_B, _D, 24, 24)
    indices = idx3.reshape(_B, 1, 24, 24)
    loss = (loss_acc[0, 0] / jnp.float32(_N * _D)) * jnp.float32(1.25)
    return quantized, indices, loss


# speedup vs baseline: 1.0564x; 1.0096x over previous
import functools

import jax
import jax.numpy as jnp
from jax import lax
from jax.experimental import pallas as pl
from jax.experimental.pallas import tpu as pltpu
from jax.experimental.pallas import tpu_sc as plsc

_NE = 1024
_DP = 128
_D = 64
_HW = 576
_B = 64
_N = _B * _HW

_NC = 2
_NS = 16
_NW = _NC * _NS
_BPT = _B // _NW
_RG = 4
_RPS = _RG * _HW


def _vq_body(z_ref, emb_ref, embm2_ref, idx_ref, loss_ref):
    x = z_ref[...].reshape(_RPS, _D)
    emb = emb_ref[...]
    a2 = jnp.sum(x * x, axis=1, keepdims=True)
    b2 = jnp.sum(emb * emb, axis=1)[None, :]
    ab2 = lax.dot_general(x, embm2_ref[...], (((1,), (1,)), ((), ())),
                          preferred_element_type=jnp.float32)
    sq = (a2 + b2) + ab2
    d = jnp.sqrt(jnp.maximum(sq, 0.0))
    dmin = jnp.min(d, axis=1, keepdims=True)
    ji = lax.broadcasted_iota(jnp.int32, (_RPS, _NE), 1)
    idx = jnp.min(jnp.where(d == dmin, ji, jnp.int32(2**30)), axis=1)
    idx_ref[0, 0, :] = idx
    part = jnp.sum(dmin * dmin)
    @pl.when(pl.program_id(0) == 0)
    def _():
        loss_ref[0, 0] = 0.0
    loss_ref[0, 0] += part


_sc_mesh = plsc.VectorSubcoreMesh(core_axis_name="c", subcore_axis_name="s")


@functools.partial(
    pl.kernel,
    mesh=_sc_mesh,
    out_type=jax.ShapeDtypeStruct((_B, _HW, _DP), jnp.float32),
    scratch_types=[
        pltpu.VMEM((_HW,), jnp.int32),
        pltpu.VMEM((_HW, _DP), jnp.float32),
        pltpu.SemaphoreType.DMA,
    ],
)
def _gather_rows(idx_hbm, tab_hbm, out_hbm, idx_v, rows_v, sem):
    wid = lax.axis_index("s") * _NC + lax.axis_index("c")
    for bl in range(_BPT):
        b = wid * _BPT + bl
        pltpu.sync_copy(idx_hbm.at[pl.ds(b * _HW, _HW)], idx_v)
        pltpu.async_copy(tab_hbm.at[idx_v], rows_v, sem).wait()
        pltpu.sync_copy(rows_v, out_hbm.at[b])


@jax.jit
def kernel(z, embeddings):
    zf = z.reshape(_B // _RG, _RPS, _D)
    idx3, loss_acc = pl.pallas_call(
        _vq_body,
        grid=(_B // _RG,),
        in_specs=[
            pl.BlockSpec((1, _RPS, _D), lambda i: (i, 0, 0)),
            pl.BlockSpec((_NE, _D), lambda i: (0, 0)),
            pl.BlockSpec((_NE, _D), lambda i: (0, 0)),
        ],
        out_specs=[
            pl.BlockSpec((1, 1, _RPS), lambda i: (i, 0, 0)),
            pl.BlockSpec(memory_space=pltpu.SMEM, block_shape=(1, 1),
                         index_map=lambda i: (0, 0)),
        ],
        out_shape=[
            jax.ShapeDtypeStruct((_B // _RG, 1, _RPS), jnp.int32),
            jax.ShapeDtypeStruct((1, 1), jnp.float32),
        ],
        compiler_params=pltpu.CompilerParams(
            dimension_semantics=("arbitrary",)),
    )(zf, embeddings, embeddings * jnp.float32(-2.0))
    tab_pad = jnp.concatenate(
        [embeddings, jnp.zeros((_NE, _DP - _D), jnp.float32)], axis=1)
    qp = _gather_rows(idx3.reshape(_N), tab_pad)
    quantized = qp[:, :, :_D].transpose(0, 2, 1).reshape(_B, _D, 24, 24)
    indices = idx3.reshape(_B, 1, 24, 24)
    loss = (loss_acc[0, 0] / jnp.float32(_N * _D)) * jnp.float32(1.25)
    return quantized, indices, loss
